# baseline (speedup 1.0000x reference)
import functools

import jax
import jax.numpy as jnp
from jax import lax
from jax.experimental import pallas as pl
from jax.experimental.pallas import tpu as pltpu
from jax.experimental.pallas import tpu_sc as plsc

_L = 16
_NW = 32
_PB = 128
_LN2 = 0.6931471805599453
_SQRT2 = 1.4142135623730951


def _bcast_last(v):
    idx = jnp.full((_L, 1), _L - 1, jnp.int32)
    dn = lax.GatherDimensionNumbers(
        offset_dims=(), collapsed_slice_dims=(0,), start_index_map=(0,))
    return lax.gather(v, idx, dn, (1,),
                      mode=lax.GatherScatterMode.PROMISE_IN_BOUNDS)


def _neg_log_sigmoid(z):
    sig = 1.0 / (1.0 + jnp.exp(-z))
    t = sig + 1e-10
    bits = lax.bitcast_convert_type(t, jnp.int32)
    e = lax.shift_right_arithmetic(bits, 23) - 127
    m = lax.bitcast_convert_type(
        (bits & 0x007FFFFF) | 0x3F800000, jnp.float32)
    big = m > _SQRT2
    m = jnp.where(big, m * 0.5, m)
    ef = e.astype(jnp.float32) + jnp.where(big, 1.0, 0.0)
    s = (m - 1.0) / (m + 1.0)
    s2 = s * s
    logm = 2.0 * s * (1.0 + s2 * (1.0 / 3.0 + s2 * (0.2 + s2 * (1.0 / 7.0 + s2 / 9.0))))
    return -(ef * _LN2 + logm)


def _sc_body(heads_hbm, pos_hbm, neg_hbm, eemb_hbm, remb_hbm, out_hbm,
             idx_h, idx_p, idx_n, h_rows, p_rows, n_rows, out_v, sem,
             *, groups):
    wid = lax.axis_index("s") * 2 + lax.axis_index("c")

    pltpu.sync_copy(heads_hbm.at[wid], idx_h)
    pltpu.sync_copy(pos_hbm.at[wid], idx_p)
    pltpu.sync_copy(neg_hbm.at[wid], idx_n)

    copies = []
    for k in range(groups):
        dst = pl.ds(k * _PB, _PB)
        copies.append(pltpu.async_copy(eemb_hbm.at[idx_h.at[k]], h_rows.at[dst], sem))
        copies.append(pltpu.async_copy(remb_hbm.at[idx_p.at[k]], p_rows.at[dst], sem))
        copies.append(pltpu.async_copy(remb_hbm.at[idx_n.at[k]], n_rows.at[dst], sem))
    for c in copies:
        c.wait()

    n_steps = (groups * _PB) // _L
    lanes = lax.iota(jnp.int32, _L)

    def step(g, acc):
        base = g * _L
        z = jnp.zeros((_L,), jnp.float32)
        for j in range(_L):
            r = base + j
            prod = h_rows[r] * (p_rows[r] - n_rows[r])
            total = _bcast_last(plsc.cumsum(prod))
            z = jnp.where(lanes == j, total, z)
        return acc + _neg_log_sigmoid(z)

    acc = lax.fori_loop(0, n_steps, step, jnp.zeros((_L,), jnp.float32))
    out_v[...] = acc
    pltpu.sync_copy(out_v, out_hbm.at[wid])


def _tc_mean(x_ref, o_ref, *, inv_b):
    o_ref[0, 0] = jnp.sum(x_ref[...]) * inv_b


def kernel(heads, pos_rels, neg_rels, entity_emb, relation_emb):
    b = heads.shape[0]
    assert b % (_NW * _PB) == 0
    groups = b // (_NW * _PB)

    h3 = heads.astype(jnp.int32).reshape(_NW, groups, _PB)
    p3 = pos_rels.astype(jnp.int32).reshape(_NW, groups, _PB)
    n3 = neg_rels.astype(jnp.int32).reshape(_NW, groups, _PB)
    eemb = entity_emb.astype(jnp.float32)
    remb = relation_emb.astype(jnp.float32)

    bw = groups * _PB
    mesh = plsc.VectorSubcoreMesh(core_axis_name="c", subcore_axis_name="s")
    sc = pl.kernel(
        functools.partial(_sc_body, groups=groups),
        out_type=jax.ShapeDtypeStruct((_NW, _L), jnp.float32),
        mesh=mesh,
        compiler_params=pltpu.CompilerParams(
            needs_layout_passes=False, use_tc_tiling_on_sc=False),
        scratch_types=[
            pltpu.VMEM((groups, _PB), jnp.int32),
            pltpu.VMEM((groups, _PB), jnp.int32),
            pltpu.VMEM((groups, _PB), jnp.int32),
            pltpu.VMEM((bw, _L), jnp.float32),
            pltpu.VMEM((bw, _L), jnp.float32),
            pltpu.VMEM((bw, _L), jnp.float32),
            pltpu.VMEM((_L,), jnp.float32),
            pltpu.SemaphoreType.DMA,
        ],
    )
    partials = sc(h3, p3, n3, eemb, remb)

    loss = pl.pallas_call(
        functools.partial(_tc_mean, inv_b=1.0 / b),
        out_shape=jax.ShapeDtypeStruct((1, 1), jnp.float32),
        out_specs=pl.BlockSpec(memory_space=pltpu.SMEM),
    )(partials)
    return loss[0, 0]


# speedup vs baseline: 3.5851x; 3.5851x over previous
import functools

import jax
import jax.numpy as jnp
from jax import lax
from jax.experimental import pallas as pl
from jax.experimental.pallas import tpu as pltpu
from jax.experimental.pallas import tpu_sc as plsc

_L = 16
_NW = 32
_CH = 64
_BW = 512
_BPC = _CH // _L
_NC = _BW // _CH
_LN2 = 0.6931471805599453
_SQRT2 = 1.4142135623730951


def _bcast_last(v):
    idx = jnp.full((_L, 1), _L - 1, jnp.int32)
    dn = lax.GatherDimensionNumbers(
        offset_dims=(), collapsed_slice_dims=(0,), start_index_map=(0,))
    return lax.gather(v, idx, dn, (1,),
                      mode=lax.GatherScatterMode.PROMISE_IN_BOUNDS)


def _neg_log_sigmoid(z):
    sig = 1.0 / (1.0 + jnp.exp(-z))
    t = sig + 1e-10
    bits = lax.bitcast_convert_type(t, jnp.int32)
    e = lax.shift_right_arithmetic(bits, 23) - 127
    m = lax.bitcast_convert_type(
        (bits & 0x007FFFFF) | 0x3F800000, jnp.float32)
    big = m > _SQRT2
    m = jnp.where(big, m * 0.5, m)
    ef = e.astype(jnp.float32) + jnp.where(big, 1.0, 0.0)
    s = (m - 1.0) / (m + 1.0)
    s2 = s * s
    logm = 2.0 * s * (1.0 + s2 * (1.0 / 3.0 + s2 * (0.2 + s2 * (1.0 / 7.0 + s2 / 9.0))))
    return -(ef * _LN2 + logm)


def _sc_body(heads_hbm, pos_hbm, neg_hbm, et_hbm, rel128_hbm, out_hbm,
             ih, ipb, inb, ips, ins, eblk, prow, nrow, out_v,
             sem_e0, sem_e1, sem_r0, sem_r1):
    wid = lax.axis_index("s") * 2 + lax.axis_index("c")
    base = wid * _BW
    lanes = lax.iota(jnp.int32, _L)

    pltpu.sync_copy(heads_hbm.at[pl.ds(base, _BW)], ih)
    pltpu.sync_copy(pos_hbm.at[pl.ds(base, _BW)], ips)
    pltpu.sync_copy(neg_hbm.at[pl.ds(base, _BW)], ins)

    def pre(t, carry):
        o = pl.ds(t * _L, _L)
        ipb[o] = lax.shift_right_logical(ips[o], 3)
        inb[o] = lax.shift_right_logical(ins[o], 3)
        return carry

    lax.fori_loop(0, _BW // _L, pre, 0)

    sems_e = (sem_e0, sem_e1)
    sems_r = (sem_r0, sem_r1)

    def fire_rel(c, rb):
        co = pl.ds(c * _CH, _CH)
        pltpu.async_copy(rel128_hbm.at[ipb.at[co]], prow.at[rb], sems_r[rb])
        pltpu.async_copy(rel128_hbm.at[inb.at[co]], nrow.at[rb], sems_r[rb])

    def drain_rel(rb):
        for _ in range(2):
            pltpu.make_async_copy(
                rel128_hbm.at[ipb.at[pl.ds(0, _CH)]], prow.at[0],
                sems_r[rb]).wait()

    def fire_entity(b, bank):
        v = ih[pl.ds(b * _L, _L)]
        for jj in range(_L):
            cb = lax.shift_right_logical(v[jj], 7)
            off = pl.multiple_of(cb * 128, 128)
            pltpu.async_copy(et_hbm.at[:, pl.ds(off, 128)],
                             eblk.at[bank * _L + jj], sems_e[bank])

    def drain_entity(bank):
        for _ in range(_L):
            pltpu.make_async_copy(
                et_hbm.at[:, pl.ds(0, 128)], eblk.at[0], sems_e[bank]).wait()

    def consume(b, bank, rb, acc):
        vh = ih[pl.ds(b * _L, _L)]
        vp = ips[pl.ds(b * _L, _L)]
        vn = ins[pl.ds(b * _L, _L)]
        jrow = (b % _BPC) * _L
        z = jnp.zeros((_L,), jnp.float32)
        for jj in range(_L):
            r = vh[jj] & 127
            hj = plsc.load_gather(eblk.at[bank * _L + jj],
                                  [lanes, jnp.zeros((_L,), jnp.int32) + r])
            jcol = jnp.full((_L,), jrow + jj, jnp.int32)
            sp = (vp[jj] & 7) * _L
            pj = plsc.load_gather(prow.at[rb], [jcol, lanes + sp])
            sn = (vn[jj] & 7) * _L
            nj = plsc.load_gather(nrow.at[rb], [jcol, lanes + sn])
            tot = _bcast_last(plsc.cumsum(hj * (pj - nj)))
            z = jnp.where(lanes == jj, tot, z)
        return acc + _neg_log_sigmoid(z)

    acc = jnp.zeros((_L,), jnp.float32)
    fire_rel(0, 0)
    fire_entity(0, 0)
    for c in range(_NC):
        rb = c % 2
        if c + 1 < _NC:
            fire_rel(c + 1, 1 - rb)
        drain_rel(rb)

        def pair(i2, acc, _c=c, _rb=rb):
            b0 = _c * _BPC + i2 * 2
            fire_entity(b0 + 1, 1)
            drain_entity(0)
            acc = consume(b0, 0, _rb, acc)

            @pl.when(b0 + 2 < _BW // _L)
            def _():
                fire_entity(b0 + 2, 0)

            drain_entity(1)
            return consume(b0 + 1, 1, _rb, acc)

        acc = lax.fori_loop(0, _BPC // 2, pair, acc)

    out_v[...] = acc
    pltpu.sync_copy(out_v, out_hbm.at[wid])


def _tc_mean(x_ref, o_ref, *, inv_b):
    o_ref[0, 0] = jnp.sum(x_ref[...]) * inv_b


def kernel(heads, pos_rels, neg_rels, entity_emb, relation_emb):
    b = heads.shape[0]
    assert b == _NW * _BW
    nrel = relation_emb.shape[0]

    ih = heads.astype(jnp.int32)
    ip = pos_rels.astype(jnp.int32)
    incs = neg_rels.astype(jnp.int32)
    et = entity_emb.T
    rel128 = relation_emb.reshape(nrel // 8, 128)

    mesh = plsc.VectorSubcoreMesh(core_axis_name="c", subcore_axis_name="s")
    sc = pl.kernel(
        _sc_body,
        out_type=jax.ShapeDtypeStruct((_NW, _L), jnp.float32),
        mesh=mesh,
        compiler_params=pltpu.CompilerParams(needs_layout_passes=False),
        scratch_types=[
            pltpu.VMEM((_BW,), jnp.int32),
            pltpu.VMEM((_BW,), jnp.int32),
            pltpu.VMEM((_BW,), jnp.int32),
            pltpu.VMEM((_BW,), jnp.int32),
            pltpu.VMEM((_BW,), jnp.int32),
            pltpu.VMEM((2 * _L, _L, 128), jnp.float32),
            pltpu.VMEM((2, _CH, 128), jnp.float32),
            pltpu.VMEM((2, _CH, 128), jnp.float32),
            pltpu.VMEM((_L,), jnp.float32),
            pltpu.SemaphoreType.DMA,
            pltpu.SemaphoreType.DMA,
            pltpu.SemaphoreType.DMA,
            pltpu.SemaphoreType.DMA,
        ],
    )
    partials = sc(ih, ip, incs, et, rel128)

    loss = pl.pallas_call(
        functools.partial(_tc_mean, inv_b=1.0 / b),
        out_shape=jax.ShapeDtypeStruct((1, 1), jnp.float32),
        out_specs=pl.BlockSpec(memory_space=pltpu.SMEM),
    )(partials)
    return loss[0, 0]


# speedup vs baseline: 3.6242x; 1.0109x over previous
import functools

import jax
import jax.numpy as jnp
from jax import lax
from jax.experimental import pallas as pl
from jax.experimental.pallas import tpu as pltpu
from jax.experimental.pallas import tpu_sc as plsc

_L = 16
_NW = 32
_CH = 64
_BW = 512
_BPC = _CH // _L
_NC = _BW // _CH
_LN2 = 0.6931471805599453
_SQRT2 = 1.4142135623730951


def _bcast_last(v):
    idx = jnp.full((_L, 1), _L - 1, jnp.int32)
    dn = lax.GatherDimensionNumbers(
        offset_dims=(), collapsed_slice_dims=(0,), start_index_map=(0,))
    return lax.gather(v, idx, dn, (1,),
                      mode=lax.GatherScatterMode.PROMISE_IN_BOUNDS)


def _neg_log_sigmoid(z):
    sig = 1.0 / (1.0 + jnp.exp(-z))
    t = sig + 1e-10
    bits = lax.bitcast_convert_type(t, jnp.int32)
    e = lax.shift_right_arithmetic(bits, 23) - 127
    m = lax.bitcast_convert_type(
        (bits & 0x007FFFFF) | 0x3F800000, jnp.float32)
    big = m > _SQRT2
    m = jnp.where(big, m * 0.5, m)
    ef = e.astype(jnp.float32) + jnp.where(big, 1.0, 0.0)
    s = (m - 1.0) / (m + 1.0)
    s2 = s * s
    logm = 2.0 * s * (1.0 + s2 * (1.0 / 3.0 + s2 * (0.2 + s2 * (1.0 / 7.0 + s2 / 9.0))))
    return -(ef * _LN2 + logm)


def _sc_body(heads_hbm, pos_hbm, neg_hbm, et_hbm, rel128_hbm, out_hbm,
             ih, ipb, inb, ips, ins, eblk, prow, nrow, out_v,
             sem_e0, sem_e1, sem_r0, sem_r1):
    wid = lax.axis_index("s") * 2 + lax.axis_index("c")
    base = wid * _BW
    lanes = lax.iota(jnp.int32, _L)

    c_ih = pltpu.async_copy(heads_hbm.at[pl.ds(base, _BW)], ih, sem_e0)
    c_ip = pltpu.async_copy(pos_hbm.at[pl.ds(base, _BW)], ips, sem_e0)
    c_in = pltpu.async_copy(neg_hbm.at[pl.ds(base, _BW)], ins, sem_e0)
    c_ih.wait()
    c_ip.wait()
    c_in.wait()

    def pre(t, carry):
        o = pl.ds(t * _L, _L)
        ipb[o] = lax.shift_right_logical(ips[o], 3)
        inb[o] = lax.shift_right_logical(ins[o], 3)
        return carry

    lax.fori_loop(0, _BW // _L, pre, 0)

    sems_e = (sem_e0, sem_e1)
    sems_r = (sem_r0, sem_r1)

    def fire_rel(c, rb):
        co = pl.ds(c * _CH, _CH)
        pltpu.async_copy(rel128_hbm.at[ipb.at[co]], prow.at[rb], sems_r[rb])
        pltpu.async_copy(rel128_hbm.at[inb.at[co]], nrow.at[rb], sems_r[rb])

    def drain_rel(rb):
        for _ in range(2):
            pltpu.make_async_copy(
                rel128_hbm.at[ipb.at[pl.ds(0, _CH)]], prow.at[0],
                sems_r[rb]).wait()

    def fire_entity(b, bank):
        v = ih[pl.ds(b * _L, _L)]
        for jj in range(_L):
            cb = lax.shift_right_logical(v[jj], 7)
            off = pl.multiple_of(cb * 128, 128)
            pltpu.async_copy(et_hbm.at[:, pl.ds(off, 128)],
                             eblk.at[bank * _L + jj], sems_e[bank])

    def drain_entity(bank):
        for _ in range(_L):
            pltpu.make_async_copy(
                et_hbm.at[:, pl.ds(0, 128)], eblk.at[0], sems_e[bank]).wait()

    def consume(b, bank, rb, acc):
        vh = ih[pl.ds(b * _L, _L)]
        vp = ips[pl.ds(b * _L, _L)]
        vn = ins[pl.ds(b * _L, _L)]
        jrow = (b % _BPC) * _L
        z = jnp.zeros((_L,), jnp.float32)
        for jj in range(_L):
            r = vh[jj] & 127
            hj = plsc.load_gather(eblk.at[bank * _L + jj],
                                  [lanes, jnp.zeros((_L,), jnp.int32) + r])
            jcol = jnp.full((_L,), jrow + jj, jnp.int32)
            sp = (vp[jj] & 7) * _L
            pj = plsc.load_gather(prow.at[rb], [jcol, lanes + sp])
            sn = (vn[jj] & 7) * _L
            nj = plsc.load_gather(nrow.at[rb], [jcol, lanes + sn])
            tot = _bcast_last(plsc.cumsum(hj * (pj - nj)))
            z = jnp.where(lanes == jj, tot, z)
        return acc + _neg_log_sigmoid(z)

    acc = jnp.zeros((_L,), jnp.float32)
    fire_rel(0, 0)
    fire_entity(0, 0)
    for c in range(_NC):
        rb = c % 2
        if c + 1 < _NC:
            fire_rel(c + 1, 1 - rb)
        drain_rel(rb)

        def pair(i2, acc, _c=c, _rb=rb):
            b0 = _c * _BPC + i2 * 2
            fire_entity(b0 + 1, 1)
            drain_entity(0)
            acc = consume(b0, 0, _rb, acc)

            @pl.when(b0 + 2 < _BW // _L)
            def _():
                fire_entity(b0 + 2, 0)

            drain_entity(1)
            return consume(b0 + 1, 1, _rb, acc)

        acc = lax.fori_loop(0, _BPC // 2, pair, acc)

    out_v[...] = acc
    pltpu.sync_copy(out_v, out_hbm.at[wid])


def _tc_mean(x_ref, o_ref, *, inv_b):
    o_ref[0, 0] = jnp.sum(x_ref[...]) * inv_b


def kernel(heads, pos_rels, neg_rels, entity_emb, relation_emb):
    b = heads.shape[0]
    assert b == _NW * _BW
    nrel = relation_emb.shape[0]

    ih = heads.astype(jnp.int32)
    ip = pos_rels.astype(jnp.int32)
    incs = neg_rels.astype(jnp.int32)
    et = entity_emb.T
    rel128 = relation_emb.reshape(nrel // 8, 128)

    mesh = plsc.VectorSubcoreMesh(core_axis_name="c", subcore_axis_name="s")
    sc = pl.kernel(
        _sc_body,
        out_type=jax.ShapeDtypeStruct((_NW, _L), jnp.float32),
        mesh=mesh,
        compiler_params=pltpu.CompilerParams(needs_layout_passes=False),
        scratch_types=[
            pltpu.VMEM((_BW,), jnp.int32),
            pltpu.VMEM((_BW,), jnp.int32),
            pltpu.VMEM((_BW,), jnp.int32),
            pltpu.VMEM((_BW,), jnp.int32),
            pltpu.VMEM((_BW,), jnp.int32),
            pltpu.VMEM((2 * _L, _L, 128), jnp.float32),
            pltpu.VMEM((2, _CH, 128), jnp.float32),
            pltpu.VMEM((2, _CH, 128), jnp.float32),
            pltpu.VMEM((_L,), jnp.float32),
            pltpu.SemaphoreType.DMA,
            pltpu.SemaphoreType.DMA,
            pltpu.SemaphoreType.DMA,
            pltpu.SemaphoreType.DMA,
        ],
    )
    partials = sc(ih, ip, incs, et, rel128)

    loss = pl.pallas_call(
        functools.partial(_tc_mean, inv_b=1.0 / b),
        out_shape=jax.ShapeDtypeStruct((1, 1), jnp.float32),
        out_specs=pl.BlockSpec(memory_space=pltpu.SMEM),
    )(partials)
    return loss[0, 0]


# speedup vs baseline: 4.0006x; 1.1039x over previous
import functools

import jax
import jax.numpy as jnp
from jax import lax
from jax.experimental import pallas as pl
from jax.experimental.pallas import tpu as pltpu
from jax.experimental.pallas import tpu_sc as plsc

_L = 16
_NW = 32
_PB = 128
_BW = 512
_G = _BW // _PB
_LN2 = 0.6931471805599453
_SQRT2 = 1.4142135623730951


def _bcast_last(v):
    idx = jnp.full((_L, 1), _L - 1, jnp.int32)
    dn = lax.GatherDimensionNumbers(
        offset_dims=(), collapsed_slice_dims=(0,), start_index_map=(0,))
    return lax.gather(v, idx, dn, (1,),
                      mode=lax.GatherScatterMode.PROMISE_IN_BOUNDS)


def _neg_log_sigmoid(z):
    sig = 1.0 / (1.0 + jnp.exp(-z))
    t = sig + 1e-10
    bits = lax.bitcast_convert_type(t, jnp.int32)
    e = lax.shift_right_arithmetic(bits, 23) - 127
    m = lax.bitcast_convert_type(
        (bits & 0x007FFFFF) | 0x3F800000, jnp.float32)
    big = m > _SQRT2
    m = jnp.where(big, m * 0.5, m)
    ef = e.astype(jnp.float32) + jnp.where(big, 1.0, 0.0)
    s = (m - 1.0) / (m + 1.0)
    s2 = s * s
    logm = 2.0 * s * (1.0 + s2 * (1.0 / 3.0 + s2 * (0.2 + s2 * (1.0 / 7.0 + s2 / 9.0))))
    return -(ef * _LN2 + logm)


def _entity_body(heads_hbm, et_hbm, out_hbm, ih, eblk, hbuf, sem_e0, sem_e1):
    wid = lax.axis_index("s") * 2 + lax.axis_index("c")
    base = wid * _BW
    lanes = lax.iota(jnp.int32, _L)

    pltpu.sync_copy(heads_hbm.at[pl.ds(base, _BW)], ih)

    sems = (sem_e0, sem_e1)

    def fire(b, bank):
        v = ih[pl.ds(b * _L, _L)]
        for jj in range(_L):
            cb = lax.shift_right_logical(v[jj], 7)
            off = pl.multiple_of(cb * 128, 128)
            pltpu.async_copy(et_hbm.at[:, pl.ds(off, 128)],
                             eblk.at[bank * _L + jj], sems[bank])

    def drain(bank):
        for _ in range(_L):
            pltpu.make_async_copy(
                et_hbm.at[:, pl.ds(0, 128)], eblk.at[0], sems[bank]).wait()

    hhalf = _BW // 2

    def consume(b, bank):
        v = ih[pl.ds(b * _L, _L)]
        for jj in range(_L):
            r = v[jj] & 127
            hj = plsc.load_gather(eblk.at[bank * _L + jj],
                                  [lanes, jnp.zeros((_L,), jnp.int32) + r])
            hbuf[(b % (hhalf // _L)) * _L + jj] = hj

    n_b = _BW // _L

    fire(0, 0)

    def pair(i2, carry):
        b0 = i2 * 2
        fire(b0 + 1, 1)
        drain(0)
        consume(b0, 0)

        @pl.when(b0 + 2 < n_b)
        def _():
            fire(b0 + 2, 0)

        drain(1)
        consume(b0 + 1, 1)

        @pl.when(i2 == (n_b // 4) - 1)
        def _():
            pltpu.sync_copy(hbuf, out_hbm.at[pl.ds(base, hhalf)])
        return carry

    lax.fori_loop(0, n_b // 2, pair, 0)
    pltpu.sync_copy(hbuf, out_hbm.at[pl.ds(base + hhalf, hhalf)])


def _score_body(pos_hbm, neg_hbm, hrows_hbm, remb_hbm, out_hbm,
                idx_p, idx_n, hbuf, prows, nrows, out_v, sem):
    wid = lax.axis_index("s") * 2 + lax.axis_index("c")
    base = wid * _BW
    lanes = lax.iota(jnp.int32, _L)

    pltpu.sync_copy(pos_hbm.at[wid], idx_p)
    pltpu.sync_copy(neg_hbm.at[wid], idx_n)

    ch = pltpu.async_copy(hrows_hbm.at[pl.ds(base, _BW)], hbuf, sem)
    half = _BW // 2
    nkh = _G // 2

    def fire_half(h):
        cps = []
        for k in range(nkh):
            dst = pl.ds(k * _PB, _PB)
            cps.append(pltpu.async_copy(
                remb_hbm.at[idx_p.at[h * nkh + k]], prows.at[dst], sem))
            cps.append(pltpu.async_copy(
                remb_hbm.at[idx_n.at[h * nkh + k]], nrows.at[dst], sem))
        return cps

    cps = fire_half(0)
    ch.wait()
    for c in cps:
        c.wait()

    acc = jnp.zeros((_L,), jnp.float32)
    for h in range(2):
        def step(g, acc, _h=h):
            z = jnp.zeros((_L,), jnp.float32)
            for jj in range(_L):
                r = g * _L + jj
                tot = _bcast_last(plsc.cumsum(
                    hbuf[_h * half + r] * (prows[r] - nrows[r])))
                z = jnp.where(lanes == jj, tot, z)
            return acc + _neg_log_sigmoid(z)

        acc = lax.fori_loop(0, half // _L, step, acc)
        if h == 0:
            cps = fire_half(1)
            for c in cps:
                c.wait()
    out_v[...] = acc
    pltpu.sync_copy(out_v, out_hbm.at[wid])


def _tc_mean(x_ref, o_ref, *, inv_b):
    o_ref[0, 0] = jnp.sum(x_ref[...]) * inv_b


def kernel(heads, pos_rels, neg_rels, entity_emb, relation_emb):
    b = heads.shape[0]
    assert b == _NW * _BW

    ih = heads.astype(jnp.int32)
    p3 = pos_rels.astype(jnp.int32).reshape(_NW, _G, _PB)
    n3 = neg_rels.astype(jnp.int32).reshape(_NW, _G, _PB)
    et = entity_emb.T
    remb = relation_emb.astype(jnp.float32)

    mesh = plsc.VectorSubcoreMesh(core_axis_name="c", subcore_axis_name="s")

    entity_k = pl.kernel(
        _entity_body,
        out_type=jax.ShapeDtypeStruct((b, _L), jnp.float32),
        mesh=mesh,
        compiler_params=pltpu.CompilerParams(needs_layout_passes=False),
        scratch_types=[
            pltpu.VMEM((_BW,), jnp.int32),
            pltpu.VMEM((2 * _L, _L, 128), jnp.float32),
            pltpu.VMEM((_BW // 2, _L), jnp.float32),
            pltpu.SemaphoreType.DMA,
            pltpu.SemaphoreType.DMA,
        ],
    )
    hrows = entity_k(ih, et)

    score_k = pl.kernel(
        _score_body,
        out_type=jax.ShapeDtypeStruct((_NW, _L), jnp.float32),
        mesh=mesh,
        compiler_params=pltpu.CompilerParams(
            needs_layout_passes=False, use_tc_tiling_on_sc=False),
        scratch_types=[
            pltpu.VMEM((_G, _PB), jnp.int32),
            pltpu.VMEM((_G, _PB), jnp.int32),
            pltpu.VMEM((_BW, _L), jnp.float32),
            pltpu.VMEM((_BW // 2, _L), jnp.float32),
            pltpu.VMEM((_BW // 2, _L), jnp.float32),
            pltpu.VMEM((_L,), jnp.float32),
            pltpu.SemaphoreType.DMA,
        ],
    )
    partials = score_k(p3, n3, hrows, remb)

    loss = pl.pallas_call(
        functools.partial(_tc_mean, inv_b=1.0 / b),
        out_shape=jax.ShapeDtypeStruct((1, 1), jnp.float32),
        out_specs=pl.BlockSpec(memory_space=pltpu.SMEM),
    )(partials)
    return loss[0, 0]


# speedup vs baseline: 4.2548x; 1.0635x over previous
import functools

import jax
import jax.numpy as jnp
from jax import lax
from jax.experimental import pallas as pl
from jax.experimental.pallas import tpu as pltpu
from jax.experimental.pallas import tpu_sc as plsc

_L = 16
_NW = 32
_PB = 128
_BW = 512
_G = _BW // _PB
_LN2 = 0.6931471805599453
_SQRT2 = 1.4142135623730951


def _bcast_last(v):
    idx = jnp.full((_L, 1), _L - 1, jnp.int32)
    dn = lax.GatherDimensionNumbers(
        offset_dims=(), collapsed_slice_dims=(0,), start_index_map=(0,))
    return lax.gather(v, idx, dn, (1,),
                      mode=lax.GatherScatterMode.PROMISE_IN_BOUNDS)


def _neg_log_sigmoid(z):
    sig = 1.0 / (1.0 + jnp.exp(-z))
    t = sig + 1e-10
    bits = lax.bitcast_convert_type(t, jnp.int32)
    e = lax.shift_right_arithmetic(bits, 23) - 127
    m = lax.bitcast_convert_type(
        (bits & 0x007FFFFF) | 0x3F800000, jnp.float32)
    big = m > _SQRT2
    m = jnp.where(big, m * 0.5, m)
    ef = e.astype(jnp.float32) + jnp.where(big, 1.0, 0.0)
    s = (m - 1.0) / (m + 1.0)
    s2 = s * s
    logm = 2.0 * s * (1.0 + s2 * (1.0 / 3.0 + s2 * (0.2 + s2 * (1.0 / 7.0 + s2 / 9.0))))
    return -(ef * _LN2 + logm)


def _entity_body(heads_hbm, et_hbm, out_hbm, ih, eblk, hbuf, sem_e0, sem_e1):
    wid = lax.axis_index("s") * 2 + lax.axis_index("c")
    base = wid * _BW
    lanes = lax.iota(jnp.int32, _L)

    pltpu.sync_copy(heads_hbm.at[pl.ds(base, _BW)], ih)

    sems = (sem_e0, sem_e1)

    def fire(b, bank):
        v = ih[pl.ds(b * _L, _L)]
        for jj in range(_L):
            cb = lax.shift_right_logical(v[jj], 7)
            off = pl.multiple_of(cb * 128, 128)
            pltpu.async_copy(et_hbm.at[:, pl.ds(off, 128)],
                             eblk.at[bank * _L + jj], sems[bank])

    def drain(bank):
        for _ in range(_L):
            pltpu.make_async_copy(
                et_hbm.at[:, pl.ds(0, 128)], eblk.at[0], sems[bank]).wait()

    hhalf = _BW // 2

    def consume(b, bank):
        v = ih[pl.ds(b * _L, _L)]
        for jj in range(_L):
            r = v[jj] & 127
            hj = plsc.load_gather(eblk.at[bank * _L + jj],
                                  [lanes, jnp.zeros((_L,), jnp.int32) + r])
            hbuf[pl.ds(((b % (hhalf // _L)) * _L + jj) * _L, _L)] = hj

    n_b = _BW // _L

    fire(0, 0)

    def pair(i2, carry):
        b0 = i2 * 2
        fire(b0 + 1, 1)
        drain(0)
        consume(b0, 0)

        @pl.when(b0 + 2 < n_b)
        def _():
            fire(b0 + 2, 0)

        drain(1)
        consume(b0 + 1, 1)

        @pl.when(i2 == (n_b // 4) - 1)
        def _():
            pltpu.sync_copy(hbuf, out_hbm.at[pl.ds(base * _L, hhalf * _L)])
        return carry

    lax.fori_loop(0, n_b // 2, pair, 0)
    pltpu.sync_copy(
        hbuf, out_hbm.at[pl.ds((base + hhalf) * _L, hhalf * _L)])


def _score_body(pos_hbm, neg_hbm, hrows_hbm, remb_hbm, out_hbm,
                idx_p, idx_n, hbuf, prows, nrows, out_v, sem):
    wid = lax.axis_index("s") * 2 + lax.axis_index("c")
    base = wid * _BW
    lanes = lax.iota(jnp.int32, _L)

    pltpu.sync_copy(pos_hbm.at[wid], idx_p)
    pltpu.sync_copy(neg_hbm.at[wid], idx_n)

    ch = pltpu.async_copy(hrows_hbm.at[pl.ds(base * _L, _BW * _L)], hbuf, sem)
    half = _BW // 2
    nkh = _G // 2

    def fire_half(h):
        cps = []
        for k in range(nkh):
            dst = pl.ds(k * _PB, _PB)
            cps.append(pltpu.async_copy(
                remb_hbm.at[idx_p.at[h * nkh + k]], prows.at[dst], sem))
            cps.append(pltpu.async_copy(
                remb_hbm.at[idx_n.at[h * nkh + k]], nrows.at[dst], sem))
        return cps

    cps = fire_half(0)
    ch.wait()
    for c in cps:
        c.wait()

    acc = jnp.zeros((_L,), jnp.float32)
    for h in range(2):
        def step(g, acc, _h=h):
            z = jnp.zeros((_L,), jnp.float32)
            for jj in range(_L):
                r = g * _L + jj
                hj = hbuf[pl.ds((_h * half + r) * _L, _L)]
                tot = _bcast_last(plsc.cumsum(hj * (prows[r] - nrows[r])))
                z = jnp.where(lanes == jj, tot, z)
            return acc + _neg_log_sigmoid(z)

        acc = lax.fori_loop(0, half // _L, step, acc)
        if h == 0:
            cps = fire_half(1)
            for c in cps:
                c.wait()
    out_v[...] = acc
    pltpu.sync_copy(out_v, out_hbm.at[wid])


def _tc_mean(x_ref, o_ref, *, inv_b):
    o_ref[0, 0] = jnp.sum(x_ref[...]) * inv_b


def kernel(heads, pos_rels, neg_rels, entity_emb, relation_emb):
    b = heads.shape[0]
    assert b == _NW * _BW

    ih = heads.astype(jnp.int32)
    p3 = pos_rels.astype(jnp.int32).reshape(_NW, _G, _PB)
    n3 = neg_rels.astype(jnp.int32).reshape(_NW, _G, _PB)
    et = entity_emb.T
    remb = relation_emb.astype(jnp.float32)

    mesh = plsc.VectorSubcoreMesh(core_axis_name="c", subcore_axis_name="s")

    entity_k = pl.kernel(
        _entity_body,
        out_type=jax.ShapeDtypeStruct((b * _L,), jnp.float32),
        mesh=mesh,
        compiler_params=pltpu.CompilerParams(needs_layout_passes=False),
        scratch_types=[
            pltpu.VMEM((_BW,), jnp.int32),
            pltpu.VMEM((2 * _L, _L, 128), jnp.float32),
            pltpu.VMEM((_BW * _L // 2,), jnp.float32),
            pltpu.SemaphoreType.DMA,
            pltpu.SemaphoreType.DMA,
        ],
    )
    hrows = entity_k(ih, et)

    score_k = pl.kernel(
        _score_body,
        out_type=jax.ShapeDtypeStruct((_NW, _L), jnp.float32),
        mesh=mesh,
        compiler_params=pltpu.CompilerParams(
            needs_layout_passes=False, use_tc_tiling_on_sc=False),
        scratch_types=[
            pltpu.VMEM((_G, _PB), jnp.int32),
            pltpu.VMEM((_G, _PB), jnp.int32),
            pltpu.VMEM((_BW * _L,), jnp.float32),
            pltpu.VMEM((_BW // 2, _L), jnp.float32),
            pltpu.VMEM((_BW // 2, _L), jnp.float32),
            pltpu.VMEM((_L,), jnp.float32),
            pltpu.SemaphoreType.DMA,
        ],
    )
    partials = score_k(p3, n3, hrows, remb)

    loss = pl.pallas_call(
        functools.partial(_tc_mean, inv_b=1.0 / b),
        out_shape=jax.ShapeDtypeStruct((1, 1), jnp.float32),
        out_specs=pl.BlockSpec(memory_space=pltpu.SMEM),
    )(partials)
    return loss[0, 0]
